# baseline (speedup 1.0000x reference)
import functools

import jax
import jax.numpy as jnp
from jax import lax
from jax.experimental import pallas as pl
from jax.experimental.pallas import tpu as pltpu
from jax.experimental.pallas import tpu_sc as plsc

NC = 2
NS = 16
NW = NC * NS

BLK = 128
N_ATOMS = 100000
NB_FULL = N_ATOMS // BLK
TAIL = N_ATOMS - NB_FULL * BLK
TAIL_OFF = NB_FULL * BLK
MAX_BLOCKS_PER_W = -(-NB_FULL // NW)
STAGE = MAX_BLOCKS_PER_W * BLK


def kernel(z, tag, base_table, tag_table):
    n, d_base = N_ATOMS, base_table.shape[1]
    d_tag = tag_table.shape[1]
    d = d_base + d_tag
    zi = z.astype(jnp.int32)
    ti = tag.astype(jnp.int32)

    mesh = plsc.VectorSubcoreMesh(
        core_axis_name="c", subcore_axis_name="s",
        num_cores=NC, num_subcores=NS)

    @functools.partial(
        pl.kernel,
        out_type=jax.ShapeDtypeStruct((n, d), jnp.float32),
        mesh=mesh,
        compiler_params=pltpu.CompilerParams(use_tc_tiling_on_sc=False),
        scratch_types=[
            pltpu.VMEM((STAGE,), jnp.int32),
            pltpu.VMEM((STAGE,), jnp.int32),
            pltpu.VMEM((BLK, d_base), jnp.float32),
            pltpu.VMEM((BLK, d_tag), jnp.float32),
            pltpu.VMEM((TAIL,), jnp.int32),
            pltpu.VMEM((TAIL,), jnp.int32),
            pltpu.VMEM((TAIL, d_base), jnp.float32),
            pltpu.VMEM((TAIL, d_tag), jnp.float32),
            pltpu.SemaphoreType.DMA,
        ],
    )
    def sc_kernel(z_hbm, t_hbm, base_hbm, tagtab_hbm,
                  out_hbm, zv, tv, buf_a, buf_b, ztv, ttv, tbuf_a, tbuf_b,
                  sem):
        wid = lax.axis_index("s") * NC + lax.axis_index("c")
        lo = (wid * NB_FULL) >> 5
        hi = ((wid + 1) * NB_FULL) >> 5
        base_atom = pl.multiple_of(lo * BLK, BLK)
        pltpu.sync_copy(z_hbm.at[pl.ds(base_atom, STAGE)], zv)
        pltpu.sync_copy(t_hbm.at[pl.ds(base_atom, STAGE)], tv)

        def body(j, carry):
            off = pl.multiple_of((j - lo) * BLK, BLK)
            cp_a = pltpu.async_copy(
                base_hbm.at[zv.at[pl.ds(off, BLK)]], buf_a, sem)
            cp_b = pltpu.async_copy(
                tagtab_hbm.at[tv.at[pl.ds(off, BLK)]], buf_b, sem)
            cp_a.wait()
            cp_b.wait()
            row0 = pl.multiple_of(j * BLK, BLK)
            pltpu.sync_copy(
                buf_a, out_hbm.at[pl.ds(row0, BLK), pl.ds(0, d_base)])
            pltpu.sync_copy(
                buf_b, out_hbm.at[pl.ds(row0, BLK), pl.ds(d_base, d_tag)])
            return carry

        lax.fori_loop(lo, hi, body, 0)

        @pl.when(wid == 0)
        def _tail():
            pltpu.sync_copy(z_hbm.at[pl.ds(TAIL_OFF, TAIL)], ztv)
            pltpu.sync_copy(t_hbm.at[pl.ds(TAIL_OFF, TAIL)], ttv)
            cp_a = pltpu.async_copy(base_hbm.at[ztv], tbuf_a, sem)
            cp_b = pltpu.async_copy(tagtab_hbm.at[ttv], tbuf_b, sem)
            cp_a.wait()
            cp_b.wait()
            pltpu.sync_copy(
                tbuf_a, out_hbm.at[pl.ds(TAIL_OFF, TAIL), pl.ds(0, d_base)])
            pltpu.sync_copy(
                tbuf_b, out_hbm.at[pl.ds(TAIL_OFF, TAIL), pl.ds(d_base, d_tag)])

    return sc_kernel(zi, ti, base_table, tag_table)
